# baseline (speedup 1.0000x reference)
import jax
import jax.numpy as jnp
from jax.experimental import pallas as pl

_N = 10000
_C = 256
_K = 17
_BLOCK = 400


def _body(x_ref, w_ref, b_ref, *out_refs):
    x = x_ref[...]
    for k in range(_K):
        y = jax.lax.dot_general(
            x, w_ref[k],
            (((1,), (1,)), ((), ())),
            preferred_element_type=jnp.float32,
        )
        out_refs[k][...] = y + b_ref[k][None, :]


def kernel(inputs, weights, bias):
    grid = (_N // _BLOCK,)
    outs = pl.pallas_call(
        _body,
        grid=grid,
        in_specs=[
            pl.BlockSpec((_BLOCK, _C), lambda i: (i, 0)),
            pl.BlockSpec((_K, _C, _C), lambda i: (0, 0, 0)),
            pl.BlockSpec((_K, _C), lambda i: (0, 0)),
        ],
        out_specs=[pl.BlockSpec((_BLOCK, _C), lambda i: (i, 0))] * _K,
        out_shape=[jax.ShapeDtypeStruct((_N, _C), jnp.float32)] * _K,
    )(inputs, weights, bias)
    return tuple(outs)


# speedup vs baseline: 1.0048x; 1.0048x over previous
import jax
import jax.numpy as jnp
from jax.experimental import pallas as pl

_N = 10000
_C = 256
_K = 17
_BLOCK = 1000


def _body(x_ref, w_ref, b_ref, *out_refs):
    x = x_ref[...]
    for k in range(_K):
        y = jax.lax.dot_general(
            x, w_ref[k],
            (((1,), (1,)), ((), ())),
            preferred_element_type=jnp.float32,
        )
        out_refs[k][...] = y + b_ref[k][None, :]


def kernel(inputs, weights, bias):
    grid = (_N // _BLOCK,)
    outs = pl.pallas_call(
        _body,
        grid=grid,
        in_specs=[
            pl.BlockSpec((_BLOCK, _C), lambda i: (i, 0)),
            pl.BlockSpec((_K, _C, _C), lambda i: (0, 0, 0)),
            pl.BlockSpec((_K, _C), lambda i: (0, 0)),
        ],
        out_specs=[pl.BlockSpec((_BLOCK, _C), lambda i: (i, 0))] * _K,
        out_shape=[jax.ShapeDtypeStruct((_N, _C), jnp.float32)] * _K,
    )(inputs, weights, bias)
    return tuple(outs)
